# baseline (speedup 1.0000x reference)
import functools

import jax
import jax.numpy as jnp
import numpy as np
from jax import lax
from jax.experimental import pallas as pl
from jax.experimental.pallas import tpu as pltpu
from jax.experimental.pallas import tpu_sc as plsc

N = 10000
NP = 10240
D = 256
H = 128
E = 80000
EP = 81920

NC = 2
NS = 16
CHUNK = 128
EPT = EP // NS
NCHUNK = EPT // CHUNK
RPT = NP // NS
NROW = EP // CHUNK

_mesh = plsc.VectorSubcoreMesh(core_axis_name="c", subcore_axis_name="s")

_PERMS = np.array([[i ^ (1 << k) for i in range(16)] for k in range(4)],
                  dtype=np.int32)


@functools.partial(
    pl.kernel,
    out_type=jax.ShapeDtypeStruct((2 * NP, H), jnp.float32),
    mesh=_mesh,
    scratch_types=[
        pltpu.VMEM((NCHUNK, CHUNK), jnp.int32),
        pltpu.VMEM((NCHUNK, CHUNK), jnp.int32),
        pltpu.VMEM((CHUNK, H), jnp.float32),
        pltpu.VMEM((CHUNK, H), jnp.float32),
        pltpu.VMEM_SHARED((NP, H), jnp.float32),
        pltpu.SemaphoreType.DMA,
        pltpu.SemaphoreType.DMA,
    ],
)
def _sc_conv1(x2_hbm, src2_hbm, dst_hbm, zeros_hbm,
              agg_hbm,
              src_v, dst_v, rows0_v, rows1_v, agg_sh, sem0, sem1):
    c = lax.axis_index("c")
    s = lax.axis_index("s")

    pltpu.sync_copy(zeros_hbm.at[pl.ds(s * RPT, RPT)],
                    agg_sh.at[pl.ds(s * RPT, RPT)])
    pltpu.sync_copy(src2_hbm.at[pl.ds((c * NS + s) * NCHUNK, NCHUNK)], src_v)
    pltpu.sync_copy(dst_hbm.at[pl.ds(s * NCHUNK, NCHUNK)], dst_v)
    plsc.subcore_barrier()

    pltpu.async_copy(x2_hbm.at[src_v.at[0]], rows0_v, sem0)
    pltpu.async_copy(x2_hbm.at[src_v.at[1]], rows1_v, sem1)

    def step(jj, carry):
        j = jj * 2
        pltpu.make_async_copy(x2_hbm.at[src_v.at[j]], rows0_v, sem0).wait()
        pltpu.sync_copy(rows0_v, agg_sh.at[dst_v.at[j]], add=True)

        @pl.when(jj + 1 < NCHUNK // 2)
        def _():
            pltpu.async_copy(x2_hbm.at[src_v.at[j + 2]], rows0_v, sem0)

        pltpu.make_async_copy(x2_hbm.at[src_v.at[j + 1]], rows1_v, sem1).wait()
        pltpu.sync_copy(rows1_v, agg_sh.at[dst_v.at[j + 1]], add=True)

        @pl.when(jj + 1 < NCHUNK // 2)
        def _():
            pltpu.async_copy(x2_hbm.at[src_v.at[j + 3]], rows1_v, sem1)

        return carry

    lax.fori_loop(0, NCHUNK // 2, step, 0)
    plsc.subcore_barrier()

    pltpu.sync_copy(agg_sh.at[pl.ds(s * RPT, RPT)],
                    agg_hbm.at[pl.ds(c * NP + s * RPT, RPT)])


def _sc_conv(x2, src2, dst, zeros, dep):
    src2 = src2 + (dep * 0.0).astype(jnp.int32)
    return _sc_conv1(x2, src2, dst, zeros)


@functools.partial(
    pl.kernel,
    out_type=jax.ShapeDtypeStruct((2 * NP, H), jnp.float32),
    mesh=_mesh,
    scratch_types=[
        pltpu.VMEM((NCHUNK, CHUNK), jnp.int32),
        pltpu.VMEM((CHUNK, H), jnp.float32),
        pltpu.VMEM_SHARED((NP, H), jnp.float32),
    ],
)
def _sc_deg1(dst_hbm, zeros_hbm, ones_hbm,
             deg_hbm,
             dst_v, ones_v, deg_sh):
    c = lax.axis_index("c")
    s = lax.axis_index("s")

    pltpu.sync_copy(zeros_hbm.at[pl.ds(s * RPT, RPT)],
                    deg_sh.at[pl.ds(s * RPT, RPT)])
    pltpu.sync_copy(ones_hbm, ones_v)
    pltpu.sync_copy(dst_hbm.at[pl.ds(s * NCHUNK, NCHUNK)], dst_v)
    plsc.subcore_barrier()

    def step(j, carry):
        pltpu.sync_copy(ones_v, deg_sh.at[dst_v.at[j]], add=True)
        return carry

    lax.fori_loop(0, NCHUNK, step, 0)
    plsc.subcore_barrier()

    pltpu.sync_copy(deg_sh.at[pl.ds(s * RPT, RPT)],
                    deg_hbm.at[pl.ds(c * NP + s * RPT, RPT)])


DCH = 64
DNCH = EPT // DCH


@functools.partial(
    pl.kernel,
    out_type=jax.ShapeDtypeStruct((2 * 2 * EP,), jnp.float32),
    mesh=_mesh,
    scratch_types=[
        pltpu.VMEM((DNCH, DCH), jnp.int32),
        pltpu.VMEM((DNCH, DCH), jnp.int32),
        pltpu.VMEM((DCH, D), jnp.float32),
        pltpu.VMEM((DCH, D), jnp.float32),
        pltpu.VMEM((DCH, D), jnp.float32),
        pltpu.VMEM((DCH, D), jnp.float32),
        pltpu.VMEM((4, 16), jnp.int32),
        pltpu.VMEM((EPT,), jnp.float32),
        pltpu.SemaphoreType.DMA,
        pltpu.SemaphoreType.DMA,
        pltpu.SemaphoreType.DMA,
        pltpu.SemaphoreType.DMA,
    ],
)
def _sc_dots(h_hbm, u_hbm, v_hbm, perm_hbm, out_hbm,
             u_v, v_v, hu0_v, hv0_v, hu1_v, hv1_v, perm_v, sc_v,
             su0, sv0, su1, sv1):
    c = lax.axis_index("c")
    s = lax.axis_index("s")
    pltpu.sync_copy(perm_hbm, perm_v)
    lane = lax.iota(jnp.int32, 16)

    def compute(j, hu_v, hv_v):
        def edge(e, tot):
            a0 = hu_v[e, pl.ds(0, 16)] * hv_v[e, pl.ds(0, 16)]
            a1 = hu_v[e, pl.ds(16, 16)] * hv_v[e, pl.ds(16, 16)]
            a2 = hu_v[e, pl.ds(32, 16)] * hv_v[e, pl.ds(32, 16)]
            a3 = hu_v[e, pl.ds(48, 16)] * hv_v[e, pl.ds(48, 16)]
            for q in range(4, 16, 4):
                a0 += hu_v[e, pl.ds(q * 16, 16)] * hv_v[e, pl.ds(q * 16, 16)]
                a1 += hu_v[e, pl.ds((q + 1) * 16, 16)] * hv_v[e, pl.ds((q + 1) * 16, 16)]
                a2 += hu_v[e, pl.ds((q + 2) * 16, 16)] * hv_v[e, pl.ds((q + 2) * 16, 16)]
                a3 += hu_v[e, pl.ds((q + 3) * 16, 16)] * hv_v[e, pl.ds((q + 3) * 16, 16)]
            a = (a0 + a1) + (a2 + a3)
            for kk in range(4):
                a = a + a.at[perm_v[kk]].get(mode="promise_in_bounds",
                                             unique_indices=True)
            tot = jnp.where(lane == lax.rem(e, 16), a, tot)

            @pl.when(lax.rem(e, 16) == 15)
            def _():
                sc_v[pl.ds(j * DCH + e - 15, 16)] = tot

            return tot

        lax.fori_loop(0, DCH, edge, jnp.zeros((16,), jnp.float32))

    for t in range(2):
        base = ((t * 2 + c) * NS + s) * DNCH
        pltpu.sync_copy(u_hbm.at[pl.ds(base, DNCH)], u_v)
        pltpu.sync_copy(v_hbm.at[pl.ds(base, DNCH)], v_v)

        pltpu.async_copy(h_hbm.at[u_v.at[0]], hu0_v, su0)
        pltpu.async_copy(h_hbm.at[v_v.at[0]], hv0_v, sv0)
        pltpu.async_copy(h_hbm.at[u_v.at[1]], hu1_v, su1)
        pltpu.async_copy(h_hbm.at[v_v.at[1]], hv1_v, sv1)

        def chunk2(jj, carry):
            j = jj * 2
            pltpu.make_async_copy(h_hbm.at[u_v.at[j]], hu0_v, su0).wait()
            pltpu.make_async_copy(h_hbm.at[v_v.at[j]], hv0_v, sv0).wait()
            compute(j, hu0_v, hv0_v)

            @pl.when(jj + 1 < DNCH // 2)
            def _():
                pltpu.async_copy(h_hbm.at[u_v.at[j + 2]], hu0_v, su0)
                pltpu.async_copy(h_hbm.at[v_v.at[j + 2]], hv0_v, sv0)

            pltpu.make_async_copy(h_hbm.at[u_v.at[j + 1]], hu1_v, su1).wait()
            pltpu.make_async_copy(h_hbm.at[v_v.at[j + 1]], hv1_v, sv1).wait()
            compute(j + 1, hu1_v, hv1_v)

            @pl.when(jj + 1 < DNCH // 2)
            def _():
                pltpu.async_copy(h_hbm.at[u_v.at[j + 3]], hu1_v, su1)
                pltpu.async_copy(h_hbm.at[v_v.at[j + 3]], hv1_v, sv1)

            return carry

        lax.fori_loop(0, DNCH // 2, chunk2, 0)
        pltpu.sync_copy(sc_v, out_hbm.at[pl.ds(((t * 2 + c) * NS + s) * EPT, EPT)])


RB = 2048


def _tc_layer_body(relu, a0_ref, d0_ref, a1_ref, d1_ref, w0_ref, w1_ref, o_ref):
    d0 = jnp.maximum(d0_ref[...][:, 0:1], 1.0)
    d1 = jnp.maximum(d1_ref[...][:, 0:1], 1.0)
    x0 = jnp.concatenate([a0_ref[0], a0_ref[1]], axis=1) / d0
    x1 = jnp.concatenate([a1_ref[0], a1_ref[1]], axis=1) / d1
    out = (jnp.dot(x0, w0_ref[...], preferred_element_type=jnp.float32,
                   precision=lax.Precision.HIGHEST)
           + jnp.dot(x1, w1_ref[...], preferred_element_type=jnp.float32,
                     precision=lax.Precision.HIGHEST))
    if relu:
        out = jnp.maximum(out, 0.0)
    o_ref[...] = out


def _tc_layer1(a0, d0, a1, d1, w0, w1):
    nrb = NP // RB
    a3 = lambda: pl.BlockSpec((2, RB, H), lambda i, c: (0, i, 0))
    dsp = lambda: pl.BlockSpec((RB, 16), lambda i, c: (i, 0))
    return pl.pallas_call(
        functools.partial(_tc_layer_body, True),
        grid=(nrb, 2),
        in_specs=[a3(), dsp(), a3(), dsp(),
                  pl.BlockSpec((D, H), lambda i, c: (0, c)),
                  pl.BlockSpec((D, H), lambda i, c: (0, c))],
        out_specs=pl.BlockSpec((RB, H), lambda i, c: (c * nrb + i, 0)),
        out_shape=jax.ShapeDtypeStruct((2 * NP, H), jnp.float32),
    )(a0.reshape(2, NP, H), d0, a1.reshape(2, NP, H), d1, w0, w1)


def _tc_layer2(a0, d0, a1, d1, w0, w1):
    nrb = NP // RB
    a3 = lambda: pl.BlockSpec((2, RB, H), lambda i: (0, i, 0))
    dsp = lambda: pl.BlockSpec((RB, 16), lambda i: (i, 0))
    wsp = lambda: pl.BlockSpec((D, D), lambda i: (0, 0))
    return pl.pallas_call(
        functools.partial(_tc_layer_body, False),
        grid=(nrb,),
        in_specs=[a3(), dsp(), a3(), dsp(), wsp(), wsp()],
        out_specs=pl.BlockSpec((RB, D), lambda i: (i, 0)),
        out_shape=jax.ShapeDtypeStruct((NP, D), jnp.float32),
    )(a0.reshape(2, NP, H), d0, a1.reshape(2, NP, H), d1, w0, w1)


def _split_ei(ei):
    src = jnp.pad(ei[0].astype(jnp.int32), (0, EP - E)).reshape(NROW, CHUNK)
    dst = jnp.pad(ei[1].astype(jnp.int32), (0, EP - E),
                  constant_values=NP - 1).reshape(NROW, CHUNK)
    src2 = jnp.concatenate([src, src + NP], axis=0)
    return src2, dst


def kernel(x, block1_ei_r0, block1_ei_r1, block2_ei_r0, block2_ei_r1,
           pos_ei_r0, pos_ei_r1, neg_ei_r0, neg_ei_r1,
           W1_r0, W1_r1, W2_r0, W2_r1):
    zeros = jnp.zeros((NP, H), jnp.float32)
    ones = jnp.ones((CHUNK, H), jnp.float32)
    perms = jnp.asarray(_PERMS)

    pad = ((0, NP - N), (0, 0))
    x2 = jnp.concatenate([jnp.pad(x[:, :H], pad), jnp.pad(x[:, H:], pad)], axis=0)
    s10, d10 = _split_ei(block1_ei_r0)
    s11, d11 = _split_ei(block1_ei_r1)
    s20, d20 = _split_ei(block2_ei_r0)
    s21, d21 = _split_ei(block2_ei_r1)

    def _deg(dst, dep):
        p = _sc_deg1(dst + (dep * 0.0).astype(jnp.int32), zeros, ones)
        return p[:NP, :16]

    deg10 = _deg(d10, jnp.float32(0.0))
    deg11 = _deg(d11, deg10[0, 0])
    deg20 = _deg(d20, deg11[0, 0])
    deg21 = _deg(d21, deg20[0, 0])

    agg10 = _sc_conv(x2, s10, d10, zeros, deg21[0, 0])
    agg11 = _sc_conv(x2, s11, d11, zeros, agg10[0, 0])
    h1 = _tc_layer1(agg10, deg10, agg11, deg11, W1_r0, W1_r1)

    agg20 = _sc_conv(h1, s20, d20, zeros, h1[0, 0])
    agg21 = _sc_conv(h1, s21, d21, zeros, agg20[0, 0])
    h2 = _tc_layer2(agg20, deg20, agg21, deg21, W2_r0, W2_r1)

    def _idx(ei, row):
        return jnp.pad(ei[row].astype(jnp.int32),
                       (0, EP - E)).reshape(EP // DCH, DCH)

    u_all = jnp.concatenate([_idx(pos_ei_r0, 0), _idx(pos_ei_r1, 0),
                             _idx(neg_ei_r0, 0), _idx(neg_ei_r1, 0)], axis=0)
    v_all = jnp.concatenate([_idx(pos_ei_r0, 1), _idx(pos_ei_r1, 1),
                             _idx(neg_ei_r0, 1), _idx(neg_ei_r1, 1)], axis=0)
    scores = _sc_dots(h2, u_all, v_all, perms).reshape(2, 2, EP)
    pos_score = scores[0, :, :E].reshape(2 * E, 1)
    neg_score = scores[1, :, :E].reshape(2 * E, 1)
    return (pos_score, neg_score)
